# Initial kernel scaffold; baseline (speedup 1.0000x reference)
import jax
import jax.numpy as jnp
from jax.experimental import pallas as pl


def kernel(x, table):
    raise NotImplementedError("write your pallas kernel here")


# speedup vs baseline: 3.4613x; 3.4613x over previous
import functools
import math

import jax
import jax.numpy as jnp
from jax import lax
from jax.experimental import pallas as pl
from jax.experimental.pallas import tpu as pltpu
from jax.experimental.pallas import tpu_sc as plsc

D_EMBED = 64
VOCAB = 1000
SCALE = math.sqrt(float(D_EMBED))

NUM_CORES = 2
NUM_SUBCORES = 16
NUM_WORKERS = NUM_CORES * NUM_SUBCORES


def _scale_table_body(t_ref, o_ref):
    o_ref[...] = t_ref[...] * SCALE


@jax.jit
def _scale_table(table):
    return pl.pallas_call(
        _scale_table_body,
        out_shape=jax.ShapeDtypeStruct(table.shape, table.dtype),
    )(table)


def _make_gather(total, chunk):
    assert total % (NUM_WORKERS * chunk) == 0
    per_worker = total // NUM_WORKERS
    n_chunks = per_worker // chunk
    mesh = plsc.VectorSubcoreMesh(
        core_axis_name="c", subcore_axis_name="s",
        num_cores=NUM_CORES, num_subcores=NUM_SUBCORES,
    )

    @functools.partial(
        pl.kernel,
        out_type=jax.ShapeDtypeStruct((total, D_EMBED), jnp.float32),
        mesh=mesh,
        scratch_types=[
            pltpu.VMEM((chunk,), jnp.int32),
            pltpu.VMEM((chunk, D_EMBED), jnp.float32),
            pltpu.SemaphoreType.DMA,
        ],
        compiler_params=pltpu.CompilerParams(use_tc_tiling_on_sc=False),
    )
    def gather_kernel(idx_hbm, tbl_hbm, out_hbm, idx_v, rows_v, sem):
        wid = lax.axis_index("s") * NUM_CORES + lax.axis_index("c")
        base = wid * per_worker

        def step(i, carry):
            off = base + i * chunk
            pltpu.sync_copy(idx_hbm.at[pl.ds(off, chunk)], idx_v)
            pltpu.async_copy(tbl_hbm.at[idx_v], rows_v, sem).wait()
            pltpu.sync_copy(rows_v, out_hbm.at[pl.ds(off, chunk)])
            return carry

        lax.fori_loop(0, n_chunks, step, 0)

    return gather_kernel


_gather = _make_gather(4096 * 200, 512)


@jax.jit
def kernel(x, table):
    scaled = _scale_table(table)
    flat = x.reshape(-1)
    out = _gather(flat, scaled)
    return out.reshape(x.shape + (D_EMBED,))
